# baseline (speedup 1.0000x reference)
import functools

import jax
import jax.numpy as jnp
from jax import lax
from jax.experimental import pallas as pl
from jax.experimental.pallas import tpu as pltpu
from jax.experimental.pallas import tpu_sc as plsc

N = 10000
H = 128
E = 320000
NP = 10240
NC = 2
NS = 16
NW = NC * NS
EPT = E // NW
CH = 256
NFULL = EPT // CH
TAIL = EPT - NFULL * CH
RPT = NP // NS
DPT = NP // NS


def _sc_mesh():
    return plsc.VectorSubcoreMesh(core_axis_name="c", subcore_axis_name="s",
                                  num_cores=NC, num_subcores=NS)


def _deg_body(dst_hbm, out_hbm, dst_v, dst_t, ones_v, zer_v, acc):
    c = lax.axis_index("c")
    s = lax.axis_index("s")
    wid = c * NS + s
    for j in range(CH // 16):
        ones_v[pl.ds(j * 16, 16)] = jnp.full((16,), 1.0, jnp.float32)
    for j in range(DPT // 16):
        zer_v[pl.ds(j * 16, 16)] = jnp.zeros((16,), jnp.float32)
    pltpu.sync_copy(zer_v, acc.at[pl.ds(s * DPT, DPT)])
    plsc.subcore_barrier()
    e0 = wid * EPT

    def body(i, carry):
        base = e0 + i * CH
        pltpu.sync_copy(dst_hbm.at[pl.ds(base, CH)], dst_v)
        pltpu.sync_copy(ones_v, acc.at[dst_v], add=True)
        return carry

    lax.fori_loop(0, NFULL, body, 0)
    pltpu.sync_copy(dst_hbm.at[pl.ds(e0 + NFULL * CH, TAIL)], dst_t)
    pltpu.sync_copy(ones_v.at[pl.ds(0, TAIL)], acc.at[dst_t], add=True)
    plsc.subcore_barrier()
    pltpu.sync_copy(acc.at[pl.ds(s * DPT, DPT)],
                    out_hbm.at[pl.ds(c * NP + s * DPT, DPT)])


def _sc_deg(dst):
    k = pl.kernel(
        _deg_body,
        out_type=jax.ShapeDtypeStruct((2 * NP,), jnp.float32),
        mesh=_sc_mesh(),
        scratch_types=[
            pltpu.VMEM((CH,), jnp.int32),
            pltpu.VMEM((TAIL,), jnp.int32),
            pltpu.VMEM((CH,), jnp.float32),
            pltpu.VMEM((DPT,), jnp.float32),
            pltpu.VMEM_SHARED((NP,), jnp.float32),
        ],
    )
    return k(dst)


ACH = 128
NCHUNKS = 81
PADE = NCHUNKS * ACH - EPT
HH = H
NSEC = H // HH
NBUF = 3
ANP = 10112
ARPT = ANP // NS


def _agg_body(*refs):
    hs_hbm, sb_hbm, db_hbm, zer_hbm, out_hbm = refs[0:5]
    gidx = list(refs[5:5 + NBUF])
    sidx = list(refs[5 + NBUF:5 + 2 * NBUF])
    rows = list(refs[5 + 2 * NBUF:5 + 3 * NBUF])
    acc = refs[5 + 3 * NBUF]
    gisem = list(refs[6 + 3 * NBUF:6 + 4 * NBUF])
    sisem = list(refs[6 + 4 * NBUF:6 + 5 * NBUF])
    gsem = list(refs[6 + 5 * NBUF:6 + 6 * NBUF])
    ssem = list(refs[6 + 6 * NBUF:6 + 7 * NBUF])
    c_ax = lax.axis_index("c")
    s_ax = lax.axis_index("s")
    wid = c_ax * NS + s_ax
    e0 = wid * NCHUNKS * ACH

    pltpu.sync_copy(zer_hbm, acc.at[pl.ds(s_ax * ARPT, ARPT)])
    plsc.subcore_barrier()

    def issue_ids(c, k):
        pltpu.async_copy(sb_hbm.at[pl.ds(e0 + c * ACH, ACH)], gidx[k],
                         gisem[k])
        pltpu.async_copy(db_hbm.at[pl.ds(e0 + c * ACH, ACH)], sidx[k],
                         sisem[k])

    def wait_ids(k):
        pltpu.make_async_copy(sb_hbm.at[pl.ds(0, ACH)], gidx[k],
                              gisem[k]).wait()
        pltpu.make_async_copy(db_hbm.at[pl.ds(0, ACH)], sidx[k],
                              sisem[k]).wait()

    def issue_gather(k):
        pltpu.async_copy(hs_hbm.at[gidx[k]], rows[k], gsem[k])

    def wait_gather(k):
        pltpu.make_async_copy(hs_hbm.at[gidx[k]], rows[k], gsem[k]).wait()

    def issue_scatter(k):
        pltpu.async_copy(rows[k], acc.at[sidx[k]], ssem[k], add=True)

    def wait_scatter(k):
        pltpu.make_async_copy(rows[k], acc.at[sidx[k]], ssem[k]).wait()

    issue_ids(0, 0)
    issue_ids(1, 1)
    wait_ids(0)
    issue_gather(0)
    issue_ids(2, 2)
    wait_ids(1)
    issue_gather(1)
    wait_gather(0)
    issue_scatter(0)

    def jbody(j, carry):
        for k in range(NBUF):
            c = j * NBUF + k
            wait_scatter(k)
            issue_ids(c, k)
            kg = (k + NBUF - 1) % NBUF
            wait_ids(kg)
            issue_gather(kg)
            ks = (k + NBUF - 2) % NBUF
            wait_gather(ks)
            issue_scatter(ks)
        return carry

    lax.fori_loop(1, NCHUNKS // NBUF, jbody, 0)

    k_last = (NCHUNKS - 1) % NBUF
    wait_ids(k_last)
    issue_gather(k_last)
    wait_gather((NCHUNKS - 2) % NBUF)
    issue_scatter((NCHUNKS - 2) % NBUF)
    wait_gather(k_last)
    issue_scatter(k_last)
    for k in range(NBUF):
        wait_scatter(k)

    plsc.subcore_barrier()
    pltpu.sync_copy(acc.at[pl.ds(s_ax * ARPT, ARPT)],
                    out_hbm.at[pl.ds(c_ax * ANP + s_ax * ARPT, ARPT)])


def _sc_agg_half(hs, sb, db, zer):
    k = pl.kernel(
        _agg_body,
        out_type=jax.ShapeDtypeStruct((2 * ANP, HH), jnp.float32),
        mesh=_sc_mesh(),
        compiler_params=pltpu.CompilerParams(use_tc_tiling_on_sc=False),
        scratch_types=(
            [pltpu.VMEM((ACH,), jnp.int32)] * (2 * NBUF)
            + [pltpu.VMEM((ACH, HH), jnp.float32)] * NBUF
            + [pltpu.VMEM_SHARED((ANP, HH), jnp.float32)]
            + [pltpu.SemaphoreType.DMA] * (4 * NBUF)
        ),
    )
    return k(hs, sb, db, zer)


def _pre_body(x_ref, w_ref, b_ref, degT_ref, h_ref, inv_ref):
    y = jnp.dot(x_ref[...], w_ref[...], preferred_element_type=jnp.float32)
    y = y + b_ref[...]
    m = jnp.mean(y, axis=0, keepdims=True)
    d = y - m
    v = jnp.mean(d * d, axis=0, keepdims=True)
    h_ref[...] = jnp.maximum(d * lax.rsqrt(v + 1e-5), 0.0)
    inv_ref[...] = lax.rsqrt(degT_ref[:, 0:1] + degT_ref[:, 1:2] + 1.0)


def _pre_call(x, w, b, degT):
    return pl.pallas_call(
        _pre_body,
        out_shape=(jax.ShapeDtypeStruct((N, H), jnp.float32),
                   jax.ShapeDtypeStruct((N, 1), jnp.float32)),
    )(x, w, b, degT)


def _preagg_body(h_ref, w_ref, inv_ref, *hs_refs):
    hs = inv_ref[...] * jnp.dot(h_ref[...], w_ref[...],
                                preferred_element_type=jnp.float32)
    pad = jnp.zeros((NP - N, HH), jnp.float32)
    for i in range(NSEC):
        hs_refs[i][...] = jnp.concatenate(
            [hs[:, i * HH:(i + 1) * HH], pad], axis=0)


def _preagg_call(h, w, inv):
    return pl.pallas_call(
        _preagg_body,
        out_shape=tuple(jax.ShapeDtypeStruct((NP, HH), jnp.float32)
                        for _ in range(NSEC)),
    )(h, w, inv)


def _postagg_body(*refs):
    p_refs = refs[0:NSEC]
    hs_refs = refs[NSEC:2 * NSEC]
    inv_ref, cb_ref, cw_ref, ccb_ref, hold_ref, out_ref = refs[2 * NSEC:]
    ssum = jnp.concatenate(
        [p_refs[i][:N, :] + p_refs[i][ANP:ANP + N, :] + hs_refs[i][:N, :]
         for i in range(NSEC)], axis=1)
    agg = inv_ref[...] * ssum + cb_ref[...]
    z = jnp.dot(agg, cw_ref[...], preferred_element_type=jnp.float32)
    z = z + ccb_ref[...] + hold_ref[...]
    out_ref[...] = jnp.maximum(z, 0.0)


def _postagg_call(ps, hss, inv, cb, cw, ccb, hold):
    return pl.pallas_call(
        _postagg_body,
        out_shape=jax.ShapeDtypeStruct((N, H), jnp.float32),
    )(*ps, *hss, inv, cb, cw, ccb, hold)


def _post_body(h_ref, w_ref, b_ref, out_ref):
    h = h_ref[...]
    for i in range(3):
        y = jnp.dot(h, w_ref[i], preferred_element_type=jnp.float32)
        y = y + b_ref[i]
        m = jnp.mean(y, axis=0, keepdims=True)
        d = y - m
        v = jnp.mean(d * d, axis=0, keepdims=True)
        h = d * lax.rsqrt(v + 1e-5)
        if i != 2:
            h = jnp.maximum(h, 0.0)
    out_ref[...] = h


def _post_call(h, w, b):
    return pl.pallas_call(
        _post_body,
        out_shape=jax.ShapeDtypeStruct((N, H), jnp.float32),
    )(h, w, b)


def kernel(x_graph_1, x_graph_2, edge_index_graph_1, edge_index_graph_2,
           batch_graph_1, batch_graph_2, pre_W, pre_b, conv_W, conv_b,
           cat_W, cat_b, post_W, post_b):
    xs = [x_graph_1, x_graph_2]
    eis = [edge_index_graph_1, edge_index_graph_2]
    zer = jnp.zeros((ARPT, HH), jnp.float32)

    pad_src = jnp.zeros((NW, PADE), jnp.int32)
    pad_dst = jnp.broadcast_to(
        N + (jnp.arange(PADE, dtype=jnp.int32) % (ANP - N)), (NW, PADE))
    blocks = []
    for t in range(2):
        srcb = jnp.concatenate([eis[t][0].reshape(NW, EPT), pad_src],
                               axis=1).reshape(NW * NCHUNKS * ACH)
        dstb = jnp.concatenate([eis[t][1].reshape(NW, EPT), pad_dst],
                               axis=1).reshape(NW * NCHUNKS * ACH)
        blocks.append((srcb, dstb))

    h = [None, None]
    inv = [None, None]
    for t in range(2):
        degp = _sc_deg(eis[t][1]).reshape(2, NP)[:, :N]
        degT = degp.T
        h[t], inv[t] = _pre_call(xs[t], pre_W[-1, t],
                                 pre_b[-1, t].reshape(1, H), degT)

    L = conv_W.shape[0]
    for l in range(L):
        hs = [_preagg_call(h[t], conv_W[l, t], inv[t]) for t in range(2)]
        for t in range(2):
            parts = [_sc_agg_half(hs[t][i], blocks[t][0], blocks[t][1], zer)
                     for i in range(NSEC)]
            h[t] = _postagg_call(parts, hs[t], inv[t],
                                 conv_b[l, t].reshape(1, H), cat_W[l, t],
                                 cat_b[l, t].reshape(1, H), h[t])

    out = [None, None]
    for t in range(2):
        out[t] = _post_call(h[t], post_W[:, t], post_b[:, t].reshape(3, 1, H))
    return jnp.stack(out)


# speedup vs baseline: 1.2991x; 1.2991x over previous
import functools

import jax
import jax.numpy as jnp
from jax import lax
from jax.experimental import pallas as pl
from jax.experimental.pallas import tpu as pltpu
from jax.experimental.pallas import tpu_sc as plsc

N = 10000
H = 128
E = 320000
NP = 10240
NC = 2
NS = 16
NW = NC * NS
EPT = E // NW
CH = 256
NFULL = EPT // CH
TAIL = EPT - NFULL * CH
RPT = NP // NS
DPT = NP // NS


def _sc_mesh():
    return plsc.VectorSubcoreMesh(core_axis_name="c", subcore_axis_name="s",
                                  num_cores=NC, num_subcores=NS)


def _deg_body(dst_hbm, out_hbm, dst_v, dst_t, ones_v, zer_v, acc):
    c = lax.axis_index("c")
    s = lax.axis_index("s")
    wid = c * NS + s
    for j in range(CH // 16):
        ones_v[pl.ds(j * 16, 16)] = jnp.full((16,), 1.0, jnp.float32)
    for j in range(DPT // 16):
        zer_v[pl.ds(j * 16, 16)] = jnp.zeros((16,), jnp.float32)
    pltpu.sync_copy(zer_v, acc.at[pl.ds(s * DPT, DPT)])
    plsc.subcore_barrier()
    e0 = wid * EPT

    def body(i, carry):
        base = e0 + i * CH
        pltpu.sync_copy(dst_hbm.at[pl.ds(base, CH)], dst_v)
        pltpu.sync_copy(ones_v, acc.at[dst_v], add=True)
        return carry

    lax.fori_loop(0, NFULL, body, 0)
    pltpu.sync_copy(dst_hbm.at[pl.ds(e0 + NFULL * CH, TAIL)], dst_t)
    pltpu.sync_copy(ones_v.at[pl.ds(0, TAIL)], acc.at[dst_t], add=True)
    plsc.subcore_barrier()
    pltpu.sync_copy(acc.at[pl.ds(s * DPT, DPT)],
                    out_hbm.at[pl.ds(c * NP + s * DPT, DPT)])


def _sc_deg(dst):
    k = pl.kernel(
        _deg_body,
        out_type=jax.ShapeDtypeStruct((2 * NP,), jnp.float32),
        mesh=_sc_mesh(),
        scratch_types=[
            pltpu.VMEM((CH,), jnp.int32),
            pltpu.VMEM((TAIL,), jnp.int32),
            pltpu.VMEM((CH,), jnp.float32),
            pltpu.VMEM((DPT,), jnp.float32),
            pltpu.VMEM_SHARED((NP,), jnp.float32),
        ],
    )
    return k(dst)


ACH = 128
NCHUNKS = 80
PADE = NCHUNKS * ACH - EPT
HH = H // 2
NSEC = H // HH
NBUF = 8
G = 6
ANP = 10240
ARPT = ANP // NS


def _agg_body(*refs):
    hs_hbm, sb_hbm, db_hbm, zer_hbm, out_hbm = refs[0:5]
    src_blk, dst_blk = refs[5], refs[6]
    rows = list(refs[7:7 + NBUF])
    gidx = list(refs[7 + NBUF:7 + 2 * NBUF])
    sidx = list(refs[7 + 2 * NBUF:7 + 3 * NBUF])
    acc = refs[7 + 3 * NBUF]
    gsem = list(refs[8 + 3 * NBUF:8 + 4 * NBUF])
    ssem = list(refs[8 + 4 * NBUF:8 + 5 * NBUF])
    c_ax = lax.axis_index("c")
    s_ax = lax.axis_index("s")
    wid = c_ax * NS + s_ax

    pltpu.sync_copy(zer_hbm, acc.at[pl.ds(s_ax * ARPT, ARPT)])
    pltpu.sync_copy(sb_hbm.at[pl.ds(wid * NCHUNKS * ACH, NCHUNKS * ACH)],
                    src_blk)
    pltpu.sync_copy(db_hbm.at[pl.ds(wid * NCHUNKS * ACH, NCHUNKS * ACH)],
                    dst_blk)
    plsc.subcore_barrier()

    def widen(blk, c, dst_ref):
        for j in range(ACH // 16):
            dst_ref[pl.ds(j * 16, 16)] = blk[pl.ds(c * ACH + j * 16, 16)]

    def issue_gather(c, k):
        widen(src_blk, c, gidx[k])
        pltpu.async_copy(hs_hbm.at[gidx[k]], rows[k], gsem[k])

    def wait_gather(k):
        pltpu.make_async_copy(hs_hbm.at[gidx[k]], rows[k], gsem[k]).wait()

    def issue_scatter(c, k):
        widen(dst_blk, c, sidx[k])
        pltpu.async_copy(rows[k], acc.at[sidx[k]], ssem[k], add=True)

    def wait_scatter(k):
        pltpu.make_async_copy(rows[k], acc.at[sidx[k]], ssem[k]).wait()

    for c in range(G):
        issue_gather(c, c)

    for k in range(NBUF):
        cg = k + G
        kg = cg % NBUF
        if cg >= NBUF:
            wait_scatter(kg)
        issue_gather(cg, kg)
        wait_gather(k)
        issue_scatter(k, k)

    def jbody(j, carry):
        for k in range(NBUF):
            c = j * NBUF + k
            kg = (k + G) % NBUF
            wait_scatter(kg)
            issue_gather(c + G, kg)
            wait_gather(k)
            issue_scatter(c, k)
        return carry

    lax.fori_loop(1, NCHUNKS // NBUF - 1, jbody, 0)

    for k in range(NBUF):
        c = NCHUNKS - NBUF + k
        if c + G < NCHUNKS:
            kg = (k + G) % NBUF
            wait_scatter(kg)
            issue_gather(c + G, kg)
        wait_gather(k)
        issue_scatter(c, k)

    for k in range(NBUF):
        wait_scatter(k)

    plsc.subcore_barrier()
    pltpu.sync_copy(acc.at[pl.ds(s_ax * ARPT, ARPT)],
                    out_hbm.at[pl.ds(c_ax * ANP + s_ax * ARPT, ARPT)])


def _sc_agg_half(hs, sb, db, zer):
    k = pl.kernel(
        _agg_body,
        out_type=jax.ShapeDtypeStruct((2 * ANP, HH), jnp.float32),
        mesh=_sc_mesh(),
        compiler_params=pltpu.CompilerParams(use_tc_tiling_on_sc=False),
        scratch_types=(
            [pltpu.VMEM((NCHUNKS * ACH,), jnp.int32)] * 2
            + [pltpu.VMEM((ACH, HH), jnp.float32)] * NBUF
            + [pltpu.VMEM((ACH,), jnp.int32)] * (2 * NBUF)
            + [pltpu.VMEM_SHARED((ANP, HH), jnp.float32)]
            + [pltpu.SemaphoreType.DMA] * (2 * NBUF)
        ),
    )
    return k(hs, sb, db, zer)


def _pre_body(x_ref, w_ref, b_ref, degT_ref, h_ref, inv_ref):
    y = jnp.dot(x_ref[...], w_ref[...], preferred_element_type=jnp.float32)
    y = y + b_ref[...]
    m = jnp.mean(y, axis=0, keepdims=True)
    d = y - m
    v = jnp.mean(d * d, axis=0, keepdims=True)
    h_ref[...] = jnp.maximum(d * lax.rsqrt(v + 1e-5), 0.0)
    inv_ref[...] = lax.rsqrt(degT_ref[:, 0:1] + degT_ref[:, 1:2] + 1.0)


def _pre_call(x, w, b, degT):
    return pl.pallas_call(
        _pre_body,
        out_shape=(jax.ShapeDtypeStruct((N, H), jnp.float32),
                   jax.ShapeDtypeStruct((N, 1), jnp.float32)),
    )(x, w, b, degT)


def _preagg_body(h_ref, w_ref, inv_ref, *hs_refs):
    hs = inv_ref[...] * jnp.dot(h_ref[...], w_ref[...],
                                preferred_element_type=jnp.float32)
    pad = jnp.zeros((NP - N, HH), jnp.float32)
    for i in range(NSEC):
        hs_refs[i][...] = jnp.concatenate(
            [hs[:, i * HH:(i + 1) * HH], pad], axis=0)


def _preagg_call(h, w, inv):
    return pl.pallas_call(
        _preagg_body,
        out_shape=tuple(jax.ShapeDtypeStruct((NP, HH), jnp.float32)
                        for _ in range(NSEC)),
    )(h, w, inv)


def _postagg_body(*refs):
    p_refs = refs[0:NSEC]
    hs_refs = refs[NSEC:2 * NSEC]
    inv_ref, cb_ref, cw_ref, ccb_ref, hold_ref, out_ref = refs[2 * NSEC:]
    ssum = jnp.concatenate(
        [p_refs[i][:N, :] + p_refs[i][ANP:ANP + N, :] + hs_refs[i][:N, :]
         for i in range(NSEC)], axis=1)
    agg = inv_ref[...] * ssum + cb_ref[...]
    z = jnp.dot(agg, cw_ref[...], preferred_element_type=jnp.float32)
    z = z + ccb_ref[...] + hold_ref[...]
    out_ref[...] = jnp.maximum(z, 0.0)


def _postagg_call(ps, hss, inv, cb, cw, ccb, hold):
    return pl.pallas_call(
        _postagg_body,
        out_shape=jax.ShapeDtypeStruct((N, H), jnp.float32),
    )(*ps, *hss, inv, cb, cw, ccb, hold)


def _post_body(h_ref, w_ref, b_ref, out_ref):
    h = h_ref[...]
    for i in range(3):
        y = jnp.dot(h, w_ref[i], preferred_element_type=jnp.float32)
        y = y + b_ref[i]
        m = jnp.mean(y, axis=0, keepdims=True)
        d = y - m
        v = jnp.mean(d * d, axis=0, keepdims=True)
        h = d * lax.rsqrt(v + 1e-5)
        if i != 2:
            h = jnp.maximum(h, 0.0)
    out_ref[...] = h


def _post_call(h, w, b):
    return pl.pallas_call(
        _post_body,
        out_shape=jax.ShapeDtypeStruct((N, H), jnp.float32),
    )(h, w, b)


def kernel(x_graph_1, x_graph_2, edge_index_graph_1, edge_index_graph_2,
           batch_graph_1, batch_graph_2, pre_W, pre_b, conv_W, conv_b,
           cat_W, cat_b, post_W, post_b):
    xs = [x_graph_1, x_graph_2]
    eis = [edge_index_graph_1, edge_index_graph_2]
    zer = jnp.zeros((ARPT, HH), jnp.float32)

    pad_src = jnp.zeros((NW, PADE), jnp.int32)
    pad_dst = jnp.broadcast_to(
        N + (jnp.arange(PADE, dtype=jnp.int32) % (ANP - N)), (NW, PADE))
    blocks = []
    for t in range(2):
        srcb = jnp.concatenate([eis[t][0].reshape(NW, EPT), pad_src],
                               axis=1).reshape(NW * NCHUNKS * ACH)
        dstb = jnp.concatenate([eis[t][1].reshape(NW, EPT), pad_dst],
                               axis=1).reshape(NW * NCHUNKS * ACH)
        blocks.append((srcb, dstb))

    h = [None, None]
    inv = [None, None]
    for t in range(2):
        degp = _sc_deg(eis[t][1]).reshape(2, NP)[:, :N]
        degT = degp.T
        h[t], inv[t] = _pre_call(xs[t], pre_W[-1, t],
                                 pre_b[-1, t].reshape(1, H), degT)

    L = conv_W.shape[0]
    for l in range(L):
        hs = [_preagg_call(h[t], conv_W[l, t], inv[t]) for t in range(2)]
        for t in range(2):
            parts = [_sc_agg_half(hs[t][i], blocks[t][0], blocks[t][1], zer)
                     for i in range(NSEC)]
            h[t] = _postagg_call(parts, hs[t], inv[t],
                                 conv_b[l, t].reshape(1, H), cat_W[l, t],
                                 cat_b[l, t].reshape(1, H), h[t])

    out = [None, None]
    for t in range(2):
        out[t] = _post_call(h[t], post_W[:, t], post_b[:, t].reshape(3, 1, H))
    return jnp.stack(out)


# speedup vs baseline: 2.8573x; 2.1995x over previous
import functools

import jax
import jax.numpy as jnp
from jax import lax
from jax.experimental import pallas as pl
from jax.experimental.pallas import tpu as pltpu
from jax.experimental.pallas import tpu_sc as plsc

N = 10000
H = 128
E = 320000
NP = 10240
NC = 2
NS = 16
NW = NC * NS
EPT = E // NW
CH = 256
NFULL = EPT // CH
TAIL = EPT - NFULL * CH
RPT = NP // NS
DPT = NP // NS


def _sc_mesh():
    return plsc.VectorSubcoreMesh(core_axis_name="c", subcore_axis_name="s",
                                  num_cores=NC, num_subcores=NS)


def _deg_body(dst_hbm, out_hbm, dst_v, dst_t, ones_v, zer_v, acc):
    c = lax.axis_index("c")
    s = lax.axis_index("s")
    wid = c * NS + s
    for j in range(CH // 16):
        ones_v[pl.ds(j * 16, 16)] = jnp.full((16,), 1.0, jnp.float32)
    for j in range(DPT // 16):
        zer_v[pl.ds(j * 16, 16)] = jnp.zeros((16,), jnp.float32)
    pltpu.sync_copy(zer_v, acc.at[pl.ds(s * DPT, DPT)])
    plsc.subcore_barrier()
    e0 = wid * EPT

    def body(i, carry):
        base = e0 + i * CH
        pltpu.sync_copy(dst_hbm.at[pl.ds(base, CH)], dst_v)
        pltpu.sync_copy(ones_v, acc.at[dst_v], add=True)
        return carry

    lax.fori_loop(0, NFULL, body, 0)
    pltpu.sync_copy(dst_hbm.at[pl.ds(e0 + NFULL * CH, TAIL)], dst_t)
    pltpu.sync_copy(ones_v.at[pl.ds(0, TAIL)], acc.at[dst_t], add=True)
    plsc.subcore_barrier()
    pltpu.sync_copy(acc.at[pl.ds(s * DPT, DPT)],
                    out_hbm.at[pl.ds(c * NP + s * DPT, DPT)])


def _sc_deg(dst):
    k = pl.kernel(
        _deg_body,
        out_type=jax.ShapeDtypeStruct((2 * NP,), jnp.float32),
        mesh=_sc_mesh(),
        scratch_types=[
            pltpu.VMEM((CH,), jnp.int32),
            pltpu.VMEM((TAIL,), jnp.int32),
            pltpu.VMEM((CH,), jnp.float32),
            pltpu.VMEM((DPT,), jnp.float32),
            pltpu.VMEM_SHARED((NP,), jnp.float32),
        ],
    )
    return k(dst)


ACH = 128
NCHUNKS = 80
PADE = NCHUNKS * ACH - EPT
HH = H // 2
NSEC = H // HH
NBUF = 2
G = 1
ANP = 10240
ARPT = ANP // NS


def _agg_body(*refs):
    hs_hbm, sb_hbm, db_hbm, zer_hbm, out_hbm = refs[0:5]
    src_blk, dst_blk = refs[5], refs[6]
    rows = list(refs[7:7 + NBUF])
    gidx = list(refs[7 + NBUF:7 + 2 * NBUF])
    sidx = list(refs[7 + 2 * NBUF:7 + 3 * NBUF])
    acc = refs[7 + 3 * NBUF]
    hs_spm = refs[8 + 3 * NBUF]
    gsem = list(refs[9 + 3 * NBUF:9 + 4 * NBUF])
    ssem = list(refs[9 + 4 * NBUF:9 + 5 * NBUF])
    c_ax = lax.axis_index("c")
    s_ax = lax.axis_index("s")
    wid = c_ax * NS + s_ax

    pltpu.sync_copy(zer_hbm, acc.at[pl.ds(s_ax * ARPT, ARPT)])
    pltpu.sync_copy(sb_hbm.at[pl.ds(wid * NCHUNKS * ACH, NCHUNKS * ACH)],
                    src_blk)
    pltpu.sync_copy(db_hbm.at[pl.ds(wid * NCHUNKS * ACH, NCHUNKS * ACH)],
                    dst_blk)
    pltpu.sync_copy(hs_hbm.at[pl.ds(s_ax * ARPT, ARPT)],
                    hs_spm.at[pl.ds(s_ax * ARPT, ARPT)])
    plsc.subcore_barrier()

    def widen(blk, c, dst_ref):
        for j in range(ACH // 16):
            dst_ref[pl.ds(j * 16, 16)] = blk[pl.ds(c * ACH + j * 16, 16)]

    def issue_gather(c, k):
        widen(src_blk, c, gidx[k])
        pltpu.async_copy(hs_spm.at[gidx[k]], rows[k], gsem[k])

    def wait_gather(k):
        pltpu.make_async_copy(hs_spm.at[gidx[k]], rows[k], gsem[k]).wait()

    def issue_scatter(c, k):
        widen(dst_blk, c, sidx[k])
        pltpu.async_copy(rows[k], acc.at[sidx[k]], ssem[k], add=True)

    def wait_scatter(k):
        pltpu.make_async_copy(rows[k], acc.at[sidx[k]], ssem[k]).wait()

    for c in range(G):
        issue_gather(c, c)

    for k in range(NBUF):
        cg = k + G
        kg = cg % NBUF
        if cg >= NBUF:
            wait_scatter(kg)
        issue_gather(cg, kg)
        wait_gather(k)
        issue_scatter(k, k)

    def jbody(j, carry):
        for k in range(NBUF):
            c = j * NBUF + k
            kg = (k + G) % NBUF
            wait_scatter(kg)
            issue_gather(c + G, kg)
            wait_gather(k)
            issue_scatter(c, k)
        return carry

    lax.fori_loop(1, NCHUNKS // NBUF - 1, jbody, 0)

    for k in range(NBUF):
        c = NCHUNKS - NBUF + k
        if c + G < NCHUNKS:
            kg = (k + G) % NBUF
            wait_scatter(kg)
            issue_gather(c + G, kg)
        wait_gather(k)
        issue_scatter(c, k)

    for k in range(NBUF):
        wait_scatter(k)

    plsc.subcore_barrier()
    pltpu.sync_copy(acc.at[pl.ds(s_ax * ARPT, ARPT)],
                    out_hbm.at[pl.ds(c_ax * ANP + s_ax * ARPT, ARPT)])


def _sc_agg_half(hs, sb, db, zer):
    k = pl.kernel(
        _agg_body,
        out_type=jax.ShapeDtypeStruct((2 * ANP, HH), jnp.float32),
        mesh=_sc_mesh(),
        compiler_params=pltpu.CompilerParams(use_tc_tiling_on_sc=False),
        scratch_types=(
            [pltpu.VMEM((NCHUNKS * ACH,), jnp.int32)] * 2
            + [pltpu.VMEM((ACH, HH), jnp.float32)] * NBUF
            + [pltpu.VMEM((ACH,), jnp.int32)] * (2 * NBUF)
            + [pltpu.VMEM_SHARED((ANP, HH), jnp.float32)] * 2
            + [pltpu.SemaphoreType.DMA] * (2 * NBUF)
        ),
    )
    return k(hs, sb, db, zer)


def _pre_body(x_ref, w_ref, b_ref, degT_ref, h_ref, inv_ref):
    y = jnp.dot(x_ref[...], w_ref[...], preferred_element_type=jnp.float32)
    y = y + b_ref[...]
    m = jnp.mean(y, axis=0, keepdims=True)
    d = y - m
    v = jnp.mean(d * d, axis=0, keepdims=True)
    h_ref[...] = jnp.maximum(d * lax.rsqrt(v + 1e-5), 0.0)
    inv_ref[...] = lax.rsqrt(degT_ref[:, 0:1] + degT_ref[:, 1:2] + 1.0)


def _pre_call(x, w, b, degT):
    return pl.pallas_call(
        _pre_body,
        out_shape=(jax.ShapeDtypeStruct((N, H), jnp.float32),
                   jax.ShapeDtypeStruct((N, 1), jnp.float32)),
    )(x, w, b, degT)


def _preagg_body(h_ref, w_ref, inv_ref, *hs_refs):
    hs = inv_ref[...] * jnp.dot(h_ref[...], w_ref[...],
                                preferred_element_type=jnp.float32)
    pad = jnp.zeros((NP - N, HH), jnp.float32)
    for i in range(NSEC):
        hs_refs[i][...] = jnp.concatenate(
            [hs[:, i * HH:(i + 1) * HH], pad], axis=0)


def _preagg_call(h, w, inv):
    return pl.pallas_call(
        _preagg_body,
        out_shape=tuple(jax.ShapeDtypeStruct((NP, HH), jnp.float32)
                        for _ in range(NSEC)),
    )(h, w, inv)


def _postagg_body(*refs):
    p_refs = refs[0:NSEC]
    hs_refs = refs[NSEC:2 * NSEC]
    inv_ref, cb_ref, cw_ref, ccb_ref, hold_ref, out_ref = refs[2 * NSEC:]
    ssum = jnp.concatenate(
        [p_refs[i][:N, :] + p_refs[i][ANP:ANP + N, :] + hs_refs[i][:N, :]
         for i in range(NSEC)], axis=1)
    agg = inv_ref[...] * ssum + cb_ref[...]
    z = jnp.dot(agg, cw_ref[...], preferred_element_type=jnp.float32)
    z = z + ccb_ref[...] + hold_ref[...]
    out_ref[...] = jnp.maximum(z, 0.0)


def _postagg_call(ps, hss, inv, cb, cw, ccb, hold):
    return pl.pallas_call(
        _postagg_body,
        out_shape=jax.ShapeDtypeStruct((N, H), jnp.float32),
    )(*ps, *hss, inv, cb, cw, ccb, hold)


def _post_body(h_ref, w_ref, b_ref, out_ref):
    h = h_ref[...]
    for i in range(3):
        y = jnp.dot(h, w_ref[i], preferred_element_type=jnp.float32)
        y = y + b_ref[i]
        m = jnp.mean(y, axis=0, keepdims=True)
        d = y - m
        v = jnp.mean(d * d, axis=0, keepdims=True)
        h = d * lax.rsqrt(v + 1e-5)
        if i != 2:
            h = jnp.maximum(h, 0.0)
    out_ref[...] = h


def _post_call(h, w, b):
    return pl.pallas_call(
        _post_body,
        out_shape=jax.ShapeDtypeStruct((N, H), jnp.float32),
    )(h, w, b)


def kernel(x_graph_1, x_graph_2, edge_index_graph_1, edge_index_graph_2,
           batch_graph_1, batch_graph_2, pre_W, pre_b, conv_W, conv_b,
           cat_W, cat_b, post_W, post_b):
    xs = [x_graph_1, x_graph_2]
    eis = [edge_index_graph_1, edge_index_graph_2]
    zer = jnp.zeros((ARPT, HH), jnp.float32)

    pad_src = jnp.zeros((NW, PADE), jnp.int32)
    pad_dst = jnp.broadcast_to(
        N + (jnp.arange(PADE, dtype=jnp.int32) % (ANP - N)), (NW, PADE))
    blocks = []
    for t in range(2):
        srcb = jnp.concatenate([eis[t][0].reshape(NW, EPT), pad_src],
                               axis=1).reshape(NW * NCHUNKS * ACH)
        dstb = jnp.concatenate([eis[t][1].reshape(NW, EPT), pad_dst],
                               axis=1).reshape(NW * NCHUNKS * ACH)
        blocks.append((srcb, dstb))

    h = [None, None]
    inv = [None, None]
    for t in range(2):
        degp = _sc_deg(eis[t][1]).reshape(2, NP)[:, :N]
        degT = degp.T
        h[t], inv[t] = _pre_call(xs[t], pre_W[-1, t],
                                 pre_b[-1, t].reshape(1, H), degT)

    L = conv_W.shape[0]
    for l in range(L):
        hs = [_preagg_call(h[t], conv_W[l, t], inv[t]) for t in range(2)]
        for t in range(2):
            parts = [_sc_agg_half(hs[t][i], blocks[t][0], blocks[t][1], zer)
                     for i in range(NSEC)]
            h[t] = _postagg_call(parts, hs[t], inv[t],
                                 conv_b[l, t].reshape(1, H), cat_W[l, t],
                                 cat_b[l, t].reshape(1, H), h[t])

    out = [None, None]
    for t in range(2):
        out[t] = _post_call(h[t], post_W[:, t], post_b[:, t].reshape(3, 1, H))
    return jnp.stack(out)


# speedup vs baseline: 2.8916x; 1.0120x over previous
import functools

import jax
import jax.numpy as jnp
from jax import lax
from jax.experimental import pallas as pl
from jax.experimental.pallas import tpu as pltpu
from jax.experimental.pallas import tpu_sc as plsc

N = 10000
H = 128
E = 320000
NP = 10240
NC = 2
NS = 16
NW = NC * NS
EPT = E // NW
CH = 256
NFULL = EPT // CH
TAIL = EPT - NFULL * CH
RPT = NP // NS
DPT = NP // NS


def _sc_mesh():
    return plsc.VectorSubcoreMesh(core_axis_name="c", subcore_axis_name="s",
                                  num_cores=NC, num_subcores=NS)


def _deg_body(dst_hbm, out_hbm, dst_v, dst_t, ones_v, zer_v, acc):
    c = lax.axis_index("c")
    s = lax.axis_index("s")
    wid = c * NS + s
    for j in range(CH // 16):
        ones_v[pl.ds(j * 16, 16)] = jnp.full((16,), 1.0, jnp.float32)
    for j in range(DPT // 16):
        zer_v[pl.ds(j * 16, 16)] = jnp.zeros((16,), jnp.float32)
    pltpu.sync_copy(zer_v, acc.at[pl.ds(s * DPT, DPT)])
    plsc.subcore_barrier()
    e0 = wid * EPT

    def body(i, carry):
        base = e0 + i * CH
        pltpu.sync_copy(dst_hbm.at[pl.ds(base, CH)], dst_v)
        pltpu.sync_copy(ones_v, acc.at[dst_v], add=True)
        return carry

    lax.fori_loop(0, NFULL, body, 0)
    pltpu.sync_copy(dst_hbm.at[pl.ds(e0 + NFULL * CH, TAIL)], dst_t)
    pltpu.sync_copy(ones_v.at[pl.ds(0, TAIL)], acc.at[dst_t], add=True)
    plsc.subcore_barrier()
    pltpu.sync_copy(acc.at[pl.ds(s * DPT, DPT)],
                    out_hbm.at[pl.ds(c * NP + s * DPT, DPT)])


def _sc_deg(dst):
    k = pl.kernel(
        _deg_body,
        out_type=jax.ShapeDtypeStruct((2 * NP,), jnp.float32),
        mesh=_sc_mesh(),
        scratch_types=[
            pltpu.VMEM((CH,), jnp.int32),
            pltpu.VMEM((TAIL,), jnp.int32),
            pltpu.VMEM((CH,), jnp.float32),
            pltpu.VMEM((DPT,), jnp.float32),
            pltpu.VMEM_SHARED((NP,), jnp.float32),
        ],
    )
    return k(dst)


ACH = 128
NCHUNKS = 81
PADE = NCHUNKS * ACH - EPT
HH = H // 2
NSEC = H // HH
NBUF = 3
G = 2
ANP = 10240
ARPT = ANP // NS


def _agg_body(*refs):
    hs_hbm, sb_hbm, db_hbm, zer_hbm, out_hbm = refs[0:5]
    src_blk, dst_blk = refs[5], refs[6]
    rows = list(refs[7:7 + NBUF])
    gidx = list(refs[7 + NBUF:7 + 2 * NBUF])
    sidx = list(refs[7 + 2 * NBUF:7 + 3 * NBUF])
    acc = refs[7 + 3 * NBUF]
    hs_spm = refs[8 + 3 * NBUF]
    gsem = list(refs[9 + 3 * NBUF:9 + 4 * NBUF])
    ssem = list(refs[9 + 4 * NBUF:9 + 5 * NBUF])
    usem = list(refs[9 + 5 * NBUF:13 + 5 * NBUF])
    c_ax = lax.axis_index("c")
    s_ax = lax.axis_index("s")
    wid = c_ax * NS + s_ax

    setup = [
        (zer_hbm, acc.at[pl.ds(s_ax * ARPT, ARPT)]),
        (sb_hbm.at[pl.ds(wid * NCHUNKS * ACH, NCHUNKS * ACH)], src_blk),
        (db_hbm.at[pl.ds(wid * NCHUNKS * ACH, NCHUNKS * ACH)], dst_blk),
        (hs_hbm.at[pl.ds(s_ax * ARPT, ARPT)],
         hs_spm.at[pl.ds(s_ax * ARPT, ARPT)]),
    ]
    for i, (a, b) in enumerate(setup):
        pltpu.async_copy(a, b, usem[i])
    for i, (a, b) in enumerate(setup):
        pltpu.make_async_copy(a, b, usem[i]).wait()
    plsc.subcore_barrier()

    def widen(blk, c, dst_ref):
        for j in range(ACH // 16):
            dst_ref[pl.ds(j * 16, 16)] = blk[pl.ds(c * ACH + j * 16, 16)]

    def issue_gather(c, k):
        widen(src_blk, c, gidx[k])
        pltpu.async_copy(hs_spm.at[gidx[k]], rows[k], gsem[k])

    def wait_gather(k):
        pltpu.make_async_copy(hs_spm.at[gidx[k]], rows[k], gsem[k]).wait()

    def issue_scatter(c, k):
        widen(dst_blk, c, sidx[k])
        pltpu.async_copy(rows[k], acc.at[sidx[k]], ssem[k], add=True)

    def wait_scatter(k):
        pltpu.make_async_copy(rows[k], acc.at[sidx[k]], ssem[k]).wait()

    for c in range(G):
        issue_gather(c, c)

    for k in range(NBUF):
        cg = k + G
        kg = cg % NBUF
        if cg >= NBUF:
            wait_scatter(kg)
        issue_gather(cg, kg)
        wait_gather(k)
        issue_scatter(k, k)

    def jbody(j, carry):
        for k in range(NBUF):
            c = j * NBUF + k
            kg = (k + G) % NBUF
            wait_scatter(kg)
            issue_gather(c + G, kg)
            wait_gather(k)
            issue_scatter(c, k)
        return carry

    lax.fori_loop(1, NCHUNKS // NBUF - 1, jbody, 0)

    for k in range(NBUF):
        c = NCHUNKS - NBUF + k
        if c + G < NCHUNKS:
            kg = (k + G) % NBUF
            wait_scatter(kg)
            issue_gather(c + G, kg)
        wait_gather(k)
        issue_scatter(c, k)

    for k in range(NBUF):
        wait_scatter(k)

    plsc.subcore_barrier()
    pltpu.sync_copy(acc.at[pl.ds(s_ax * ARPT, ARPT)],
                    out_hbm.at[pl.ds(c_ax * ANP + s_ax * ARPT, ARPT)])


def _sc_agg_half(hs, sb, db, zer):
    k = pl.kernel(
        _agg_body,
        out_type=jax.ShapeDtypeStruct((2 * ANP, HH), jnp.float32),
        mesh=_sc_mesh(),
        compiler_params=pltpu.CompilerParams(use_tc_tiling_on_sc=False),
        scratch_types=(
            [pltpu.VMEM((NCHUNKS * ACH,), jnp.int32)] * 2
            + [pltpu.VMEM((ACH, HH), jnp.float32)] * NBUF
            + [pltpu.VMEM((ACH,), jnp.int32)] * (2 * NBUF)
            + [pltpu.VMEM_SHARED((ANP, HH), jnp.float32)] * 2
            + [pltpu.SemaphoreType.DMA] * (2 * NBUF + 4)
        ),
    )
    return k(hs, sb, db, zer)


def _pre_body(x_ref, w_ref, b_ref, degT_ref, h_ref, inv_ref):
    y = jnp.dot(x_ref[...], w_ref[...], preferred_element_type=jnp.float32)
    y = y + b_ref[...]
    m = jnp.mean(y, axis=0, keepdims=True)
    d = y - m
    v = jnp.mean(d * d, axis=0, keepdims=True)
    h_ref[...] = jnp.maximum(d * lax.rsqrt(v + 1e-5), 0.0)
    inv_ref[...] = lax.rsqrt(degT_ref[:, 0:1] + degT_ref[:, 1:2] + 1.0)


def _pre_call(x, w, b, degT):
    return pl.pallas_call(
        _pre_body,
        out_shape=(jax.ShapeDtypeStruct((N, H), jnp.float32),
                   jax.ShapeDtypeStruct((N, 1), jnp.float32)),
    )(x, w, b, degT)


def _preagg_body(h_ref, w_ref, inv_ref, *hs_refs):
    hs = inv_ref[...] * jnp.dot(h_ref[...], w_ref[...],
                                preferred_element_type=jnp.float32)
    pad = jnp.zeros((NP - N, HH), jnp.float32)
    for i in range(NSEC):
        hs_refs[i][...] = jnp.concatenate(
            [hs[:, i * HH:(i + 1) * HH], pad], axis=0)


def _preagg_call(h, w, inv):
    return pl.pallas_call(
        _preagg_body,
        out_shape=tuple(jax.ShapeDtypeStruct((NP, HH), jnp.float32)
                        for _ in range(NSEC)),
    )(h, w, inv)


def _postagg_body(*refs):
    p_refs = refs[0:NSEC]
    hs_refs = refs[NSEC:2 * NSEC]
    inv_ref, cb_ref, cw_ref, ccb_ref, hold_ref, out_ref = refs[2 * NSEC:]
    ssum = jnp.concatenate(
        [p_refs[i][:N, :] + p_refs[i][ANP:ANP + N, :] + hs_refs[i][:N, :]
         for i in range(NSEC)], axis=1)
    agg = inv_ref[...] * ssum + cb_ref[...]
    z = jnp.dot(agg, cw_ref[...], preferred_element_type=jnp.float32)
    z = z + ccb_ref[...] + hold_ref[...]
    out_ref[...] = jnp.maximum(z, 0.0)


def _postagg_call(ps, hss, inv, cb, cw, ccb, hold):
    return pl.pallas_call(
        _postagg_body,
        out_shape=jax.ShapeDtypeStruct((N, H), jnp.float32),
    )(*ps, *hss, inv, cb, cw, ccb, hold)


def _post_body(h_ref, w_ref, b_ref, out_ref):
    h = h_ref[...]
    for i in range(3):
        y = jnp.dot(h, w_ref[i], preferred_element_type=jnp.float32)
        y = y + b_ref[i]
        m = jnp.mean(y, axis=0, keepdims=True)
        d = y - m
        v = jnp.mean(d * d, axis=0, keepdims=True)
        h = d * lax.rsqrt(v + 1e-5)
        if i != 2:
            h = jnp.maximum(h, 0.0)
    out_ref[...] = h


def _post_call(h, w, b):
    return pl.pallas_call(
        _post_body,
        out_shape=jax.ShapeDtypeStruct((N, H), jnp.float32),
    )(h, w, b)


def kernel(x_graph_1, x_graph_2, edge_index_graph_1, edge_index_graph_2,
           batch_graph_1, batch_graph_2, pre_W, pre_b, conv_W, conv_b,
           cat_W, cat_b, post_W, post_b):
    xs = [x_graph_1, x_graph_2]
    eis = [edge_index_graph_1, edge_index_graph_2]
    zer = jnp.zeros((ARPT, HH), jnp.float32)

    pad_src = jnp.zeros((NW, PADE), jnp.int32)
    pad_dst = jnp.broadcast_to(
        N + (jnp.arange(PADE, dtype=jnp.int32) % (ANP - N)), (NW, PADE))
    blocks = []
    for t in range(2):
        srcb = jnp.concatenate([eis[t][0].reshape(NW, EPT), pad_src],
                               axis=1).reshape(NW * NCHUNKS * ACH)
        dstb = jnp.concatenate([eis[t][1].reshape(NW, EPT), pad_dst],
                               axis=1).reshape(NW * NCHUNKS * ACH)
        blocks.append((srcb, dstb))

    h = [None, None]
    inv = [None, None]
    for t in range(2):
        degp = _sc_deg(eis[t][1]).reshape(2, NP)[:, :N]
        degT = degp.T
        h[t], inv[t] = _pre_call(xs[t], pre_W[-1, t],
                                 pre_b[-1, t].reshape(1, H), degT)

    L = conv_W.shape[0]
    for l in range(L):
        hs = [_preagg_call(h[t], conv_W[l, t], inv[t]) for t in range(2)]
        for t in range(2):
            parts = [_sc_agg_half(hs[t][i], blocks[t][0], blocks[t][1], zer)
                     for i in range(NSEC)]
            h[t] = _postagg_call(parts, hs[t], inv[t],
                                 conv_b[l, t].reshape(1, H), cat_W[l, t],
                                 cat_b[l, t].reshape(1, H), h[t])

    out = [None, None]
    for t in range(2):
        out[t] = _post_call(h[t], post_W[:, t], post_b[:, t].reshape(3, 1, H))
    return jnp.stack(out)
